# baseline (speedup 1.0000x reference)
import jax
import jax.numpy as jnp
from jax.experimental import pallas as pl
from jax.experimental.pallas import tpu as pltpu

_NUM_TOKENS = 16384
_D_MODEL = 2048
_NUM_EXPERTS = 16
_BUF_T = 1024

_CHUNKS = [128, 128, 256, 512] + [1024] * 14 + [512, 256, 128, 128]
assert sum(_CHUNKS) == _NUM_TOKENS
_OFFS = [sum(_CHUNKS[:k]) for k in range(len(_CHUNKS))]


def _moe_gate_body(x_hbm, w_ref, ts_ref, ti_ref, loss_ref, b0, b1, s0, s1):
    bufs = (b0, b1)
    sems = (s0, s1)

    def copy(k):
        off, sz = _OFFS[k], _CHUNKS[k]
        return pltpu.make_async_copy(
            x_hbm.at[pl.ds(off, sz), :],
            bufs[k % 2].at[pl.ds(0, sz), :],
            sems[k % 2])

    w = w_ref[...]
    acc = jnp.zeros((_NUM_EXPERTS, 1), dtype=jnp.float32)

    copy(0).start()
    for k in range(len(_CHUNKS)):
        if k + 1 < len(_CHUNKS):
            copy(k + 1).start()
        copy(k).wait()
        off, sz = _OFFS[k], _CHUNKS[k]
        x = bufs[k % 2][pl.ds(0, sz), :]
        logits = jax.lax.dot_general(
            x, w, (((1,), (1,)), ((), ())),
            preferred_element_type=jnp.float32)
        lt = logits.T

        m = jnp.max(lt, axis=0, keepdims=True)
        e = jnp.exp(lt - m)
        s = jnp.sum(e, axis=0, keepdims=True)
        scores = e / s

        acc = acc + jnp.sum(scores, axis=1, keepdims=True)

        row = jax.lax.broadcasted_iota(jnp.int32, scores.shape, 0)
        v1 = jnp.max(scores, axis=0, keepdims=True)
        i1 = jnp.min(jnp.where(scores == v1, row, _NUM_EXPERTS),
                     axis=0, keepdims=True)
        masked = jnp.where(row == i1, -jnp.inf, scores)
        v2 = jnp.max(masked, axis=0, keepdims=True)
        i2 = jnp.min(jnp.where(masked == v2, row, _NUM_EXPERTS),
                     axis=0, keepdims=True)

        denom = v1 + v2
        ts_ref[:, pl.ds(off, sz)] = jnp.concatenate(
            [v1 / denom, v2 / denom], axis=0)
        ti_ref[:, pl.ds(off, sz)] = jnp.concatenate([i1, i2], axis=0)

    p = acc / _NUM_TOKENS
    loss_ref[0, 0] = jnp.sum(p * jnp.log(p + 1e-8))


def kernel(x, W):
    ts_t, ti_t, loss = pl.pallas_call(
        _moe_gate_body,
        in_specs=[
            pl.BlockSpec(memory_space=pl.ANY),
            pl.BlockSpec(memory_space=pltpu.VMEM),
        ],
        out_specs=[
            pl.BlockSpec(memory_space=pltpu.VMEM),
            pl.BlockSpec(memory_space=pltpu.VMEM),
            pl.BlockSpec(memory_space=pltpu.SMEM),
        ],
        out_shape=[
            jax.ShapeDtypeStruct((2, _NUM_TOKENS), jnp.float32),
            jax.ShapeDtypeStruct((2, _NUM_TOKENS), jnp.int32),
            jax.ShapeDtypeStruct((1, 1), jnp.float32),
        ],
        scratch_shapes=[
            pltpu.VMEM((_BUF_T, _D_MODEL), jnp.float32),
            pltpu.VMEM((_BUF_T, _D_MODEL), jnp.float32),
            pltpu.SemaphoreType.DMA,
            pltpu.SemaphoreType.DMA,
        ],
    )(x, W)
    return ts_t.T, ti_t.T, loss.reshape(())


# speedup vs baseline: 1.2157x; 1.2157x over previous
import jax
import jax.numpy as jnp
from jax.experimental import pallas as pl
from jax.experimental.pallas import tpu as pltpu

_NUM_TOKENS = 16384
_D_MODEL = 2048
_NUM_EXPERTS = 16
_BLOCK_T = 1024
_GRID = _NUM_TOKENS // _BLOCK_T


def _moe_gate_body(x_ref, w_ref, ts_ref, ti_ref, loss_ref, acc_ref):
    step = pl.program_id(0)

    @pl.when(step == 0)
    def _init():
        acc_ref[...] = jnp.zeros_like(acc_ref)

    w = w_ref[...]
    logits = jax.lax.dot_general(
        x_ref[...], w, (((1,), (1,)), ((), ())),
        preferred_element_type=jnp.float32)
    lt = logits.T

    m = jnp.max(lt, axis=0, keepdims=True)
    e = jnp.exp(lt - m)
    s = jnp.sum(e, axis=0, keepdims=True)
    scores = e / s

    acc_ref[...] += jnp.sum(scores, axis=1, keepdims=True)

    row = jax.lax.broadcasted_iota(jnp.int32, scores.shape, 0)
    v1 = jnp.max(scores, axis=0, keepdims=True)
    i1 = jnp.min(jnp.where(scores == v1, row, _NUM_EXPERTS),
                 axis=0, keepdims=True)
    masked = jnp.where(row == i1, -jnp.inf, scores)
    v2 = jnp.max(masked, axis=0, keepdims=True)
    i2 = jnp.min(jnp.where(masked == v2, row, _NUM_EXPERTS),
                 axis=0, keepdims=True)

    denom = v1 + v2
    ts_ref[...] = jnp.concatenate([v1 / denom, v2 / denom], axis=0)
    ti_ref[...] = jnp.concatenate([i1, i2], axis=0)

    @pl.when(step == _GRID - 1)
    def _fin():
        p = acc_ref[...] / _NUM_TOKENS
        loss_ref[0, 0] = jnp.sum(p * jnp.log(p + 1e-8))


def kernel(x, W):
    ts_t, ti_t, loss = pl.pallas_call(
        _moe_gate_body,
        grid=(_GRID,),
        in_specs=[
            pl.BlockSpec((_BLOCK_T, _D_MODEL), lambda i: (i, 0)),
            pl.BlockSpec((_NUM_EXPERTS, _D_MODEL), lambda i: (0, 0)),
        ],
        out_specs=[
            pl.BlockSpec((2, _BLOCK_T), lambda i: (0, i)),
            pl.BlockSpec((2, _BLOCK_T), lambda i: (0, i)),
            pl.BlockSpec(memory_space=pltpu.SMEM, block_shape=(1, 1),
                         index_map=lambda i: (0, 0)),
        ],
        out_shape=[
            jax.ShapeDtypeStruct((2, _NUM_TOKENS), jnp.float32),
            jax.ShapeDtypeStruct((2, _NUM_TOKENS), jnp.int32),
            jax.ShapeDtypeStruct((1, 1), jnp.float32),
        ],
        scratch_shapes=[pltpu.VMEM((_NUM_EXPERTS, 1), jnp.float32)],
        compiler_params=pltpu.CompilerParams(
            dimension_semantics=("arbitrary",)),
    )(x, W)
    return ts_t.T, ti_t.T, loss.reshape(())
